# baseline (speedup 1.0000x reference)
import functools

import jax
import jax.numpy as jnp
from jax import lax
from jax.experimental import pallas as pl
from jax.experimental.pallas import tpu as pltpu
from jax.experimental.pallas import tpu_sc as plsc

N = 50000
M = 16
F_ATOM = 64
F_NBR = 41
F_OUT = 128
EDGES = N * M
EPS = 1e-5

_B = 400
_C = 1000


def _sigmoid(x):
    return 1.0 / (1.0 + jnp.exp(-x))


def _softplus(x):
    return jnp.maximum(x, 0.0) + jnp.log(1.0 + jnp.exp(-jnp.abs(x)))


def _g_table_body(atom_ref, wnbr_ref, g_ref):
    g_ref[...] = jnp.dot(atom_ref[...], wnbr_ref[...],
                         preferred_element_type=jnp.float32)


def _g_table(atom_fea, wnbr):
    bp = 2000
    return pl.pallas_call(
        _g_table_body,
        grid=(N // bp,),
        in_specs=[
            pl.BlockSpec((bp, F_ATOM), lambda i: (i, 0)),
            pl.BlockSpec((F_ATOM, F_OUT), lambda i: (0, 0)),
        ],
        out_specs=pl.BlockSpec((bp, F_OUT), lambda i: (i, 0)),
        out_shape=jax.ShapeDtypeStruct((N, F_OUT), jnp.float32),
    )(atom_fea, wnbr)


_Q = 5
_EQ = EDGES // _Q
_NQ = N // _Q


def _sc_gather_q(g_table, idx_flat, q):
    info = plsc.get_sparse_core_info()
    nc, ns = info.num_cores, info.num_subcores
    nw = nc * ns
    bpw = _EQ // nw
    nchunk = bpw // _C
    qbase = q * _EQ
    mesh = plsc.VectorSubcoreMesh(core_axis_name="c", subcore_axis_name="s")

    @functools.partial(
        pl.kernel,
        out_type=jax.ShapeDtypeStruct((_EQ, F_OUT), jnp.float32),
        mesh=mesh,
        scratch_types=[
            pltpu.VMEM((_C,), jnp.int32),
            pltpu.VMEM((_C, F_OUT), jnp.float32),
            pltpu.SemaphoreType.DMA,
        ],
    )
    def gather_kernel(table_hbm, idx_hbm, out_hbm, idx_v, rows_v, sem):
        wid = lax.axis_index("s") * nc + lax.axis_index("c")
        base = wid * bpw
        for i in range(nchunk):
            off = base + i * _C
            pltpu.sync_copy(idx_hbm.at[pl.ds(qbase + off, _C)], idx_v)
            pltpu.async_copy(table_hbm.at[idx_v], rows_v, sem).wait()
            pltpu.sync_copy(rows_v, out_hbm.at[pl.ds(off, _C)])

    return gather_kernel(g_table, idx_flat)


def _edge_gated(gath_ref, nbr_ref, atom_ref, wself_ref, wedge_ref, bias_ref):
    r = _B * M
    s = jnp.dot(atom_ref[...], wself_ref[...],
                preferred_element_type=jnp.float32) + bias_ref[...]
    gp = gath_ref[...]
    e = jnp.dot(nbr_ref[...].reshape(r, F_NBR), wedge_ref[...],
                preferred_element_type=jnp.float32)
    return (gp + e).reshape(_B, M, F_OUT) + s[:, None, :]


def _k1_body(gath_ref, nbr_ref, atom_ref, wself_ref, wedge_ref,
             bias_ref, s1_ref, s2_ref, xh_ref):
    gated = _edge_gated(gath_ref, nbr_ref, atom_ref, wself_ref,
                        wedge_ref, bias_ref)
    g2 = gated.reshape(_B * M, F_OUT)
    xh_ref[...] = g2.astype(jnp.bfloat16)

    @pl.when(pl.program_id(0) == 0)
    def _():
        s1_ref[...] = jnp.zeros_like(s1_ref)
        s2_ref[...] = jnp.zeros_like(s2_ref)

    s1_ref[...] += jnp.sum(g2, axis=0, keepdims=True)
    s2_ref[...] += jnp.sum(g2 * g2, axis=0, keepdims=True)


def _k2_body(xh_in_ref, s1_ref, s2_ref, g1_ref, b1_ref,
             ns_ref, t1_ref, t2_ref):
    mean = s1_ref[...] / EDGES
    var = s2_ref[...] / EDGES - mean * mean
    scale = g1_ref[...] * lax.rsqrt(var + EPS)
    shift = b1_ref[...] - mean * scale

    gated = xh_in_ref[...].astype(jnp.float32).reshape(_B, M, F_OUT)
    xh = gated * scale.reshape(1, 1, F_OUT) + shift.reshape(1, 1, F_OUT)
    filt = _sigmoid(xh[..., :F_ATOM])
    core = _softplus(xh[..., F_ATOM:])
    ns = jnp.sum(filt * core, axis=1)
    ns_ref[...] = ns

    @pl.when(pl.program_id(0) == 0)
    def _():
        t1_ref[...] = jnp.zeros_like(t1_ref)
        t2_ref[...] = jnp.zeros_like(t2_ref)

    t1_ref[...] += jnp.sum(ns, axis=0, keepdims=True)
    t2_ref[...] += jnp.sum(ns * ns, axis=0, keepdims=True)


def _k3_body(atom_ref, ns_ref, t1_ref, t2_ref, g2_ref, b2_ref, out_ref):
    mean = t1_ref[...] / N
    var = t2_ref[...] / N - mean * mean
    scale = g2_ref[...] * lax.rsqrt(var + EPS)
    shift = b2_ref[...] - mean * scale
    out_ref[...] = _softplus(atom_ref[...] + ns_ref[...] * scale + shift)


def kernel(atom_fea, nbr_fea, nbr_fea_idx, W_full, b_full,
           bn1_gamma, bn1_beta, bn2_gamma, bn2_beta):
    idx_flat = nbr_fea_idx.reshape(-1).astype(jnp.int32)
    wself = W_full[:, :F_ATOM].T
    wnbr = W_full[:, F_ATOM:2 * F_ATOM].T
    wedge = W_full[:, 2 * F_ATOM:].T
    g_table = _g_table(atom_fea, wnbr)
    bias = b_full.reshape(1, F_OUT)
    g1 = bn1_gamma.reshape(1, F_OUT)
    b1 = bn1_beta.reshape(1, F_OUT)
    g2 = bn2_gamma.reshape(1, F_ATOM)
    b2 = bn2_beta.reshape(1, F_ATOM)

    gridq = (_NQ // _B,)
    nb = _NQ // _B
    const2 = lambda s: pl.BlockSpec(s, lambda i: (0, 0))

    gathered_q, k1_out = [], []
    for q in range(_Q):
        gathered_q.append(_sc_gather_q(g_table, idx_flat, q))
    for q in range(_Q):
        edge_specs = [
            pl.BlockSpec((_B * M, F_OUT), lambda i: (i, 0)),
            pl.BlockSpec((_B, M, F_NBR), lambda i, q=q: (i + nb * q, 0, 0)),
            pl.BlockSpec((_B, F_ATOM), lambda i, q=q: (i + nb * q, 0)),
            const2((F_ATOM, F_OUT)),
            const2((F_NBR, F_OUT)),
            const2((1, F_OUT)),
        ]
        k1_out.append(pl.pallas_call(
            _k1_body,
            grid=gridq,
            in_specs=edge_specs,
            out_specs=[const2((1, F_OUT)), const2((1, F_OUT)),
                       pl.BlockSpec((_B * M, F_OUT), lambda i: (i, 0))],
            out_shape=[jax.ShapeDtypeStruct((1, F_OUT), jnp.float32)] * 2
            + [jax.ShapeDtypeStruct((_EQ, F_OUT), jnp.bfloat16)],
            compiler_params=pltpu.CompilerParams(
                dimension_semantics=("arbitrary",)),
        )(gathered_q[q], nbr_fea, atom_fea, wself, wedge, bias))

    s1 = sum(o[0] for o in k1_out)
    s2 = sum(o[1] for o in k1_out)

    ns_q, t1_q, t2_q = [], [], []
    for q in range(_Q):
        nsq, t1q, t2q = pl.pallas_call(
            _k2_body,
            grid=gridq,
            in_specs=[pl.BlockSpec((_B * M, F_OUT), lambda i: (i, 0))]
            + [const2((1, F_OUT))] * 4,
            out_specs=[
                pl.BlockSpec((_B, F_ATOM), lambda i: (i, 0)),
                const2((1, F_ATOM)),
                const2((1, F_ATOM)),
            ],
            out_shape=[
                jax.ShapeDtypeStruct((_NQ, F_ATOM), jnp.float32),
                jax.ShapeDtypeStruct((1, F_ATOM), jnp.float32),
                jax.ShapeDtypeStruct((1, F_ATOM), jnp.float32),
            ],
            compiler_params=pltpu.CompilerParams(
                dimension_semantics=("arbitrary",)),
        )(k1_out[q][2], s1, s2, g1, b1)
        ns_q.append(nsq)
        t1_q.append(t1q)
        t2_q.append(t2q)

    ns = jnp.concatenate(ns_q, axis=0)
    t1 = sum(t1_q)
    t2 = sum(t2_q)

    b3 = 2000
    out = pl.pallas_call(
        _k3_body,
        grid=(N // b3,),
        in_specs=[
            pl.BlockSpec((b3, F_ATOM), lambda i: (i, 0)),
            pl.BlockSpec((b3, F_ATOM), lambda i: (i, 0)),
            const2((1, F_ATOM)),
            const2((1, F_ATOM)),
            const2((1, F_ATOM)),
            const2((1, F_ATOM)),
        ],
        out_specs=pl.BlockSpec((b3, F_ATOM), lambda i: (i, 0)),
        out_shape=jax.ShapeDtypeStruct((N, F_ATOM), jnp.float32),
        compiler_params=pltpu.CompilerParams(
            dimension_semantics=("parallel",)),
    )(atom_fea, ns, t1, t2, g2, b2)

    return out
